# baseline (speedup 1.0000x reference)
import jax
import jax.numpy as jnp
from jax import lax
from jax.experimental import pallas as pl
from jax.experimental.pallas import tpu as pltpu
from jax.experimental.pallas import tpu_sc as plsc

_N = 10000
_K = 5000
_D = 512

_NC = 2
_NS = 16
_NW = _NC * _NS
_CH = 160
_SUB = 80
_ZCH = 320
_KPAD = 5024


def _sread(ref, j):
    return ref[pl.ds(j, 16)][0]


def _unpool_body(x_hbm, idx_hbm, z_hbm, out_hbm, idxv, xv, idx_all, zbuf,
                 zlist1, zlist2, sem):
    wid = lax.axis_index("s") * _NC + lax.axis_index("c")
    lanes = lax.iota(jnp.int32, 16)

    base = jnp.minimum(wid * _CH, _K - _CH)
    pltpu.sync_copy(idx_hbm.at[pl.ds(base, _SUB)], idxv.at[0])
    pltpu.sync_copy(idx_hbm.at[pl.ds(base + _SUB, _SUB)], idxv.at[1])
    pltpu.sync_copy(x_hbm.at[pl.ds(base, _CH)], xv)
    cp0 = pltpu.async_copy(xv.at[pl.ds(0, _SUB)], out_hbm.at[idxv.at[0]], sem)
    cp1 = pltpu.async_copy(xv.at[pl.ds(_SUB, _SUB)], out_hbm.at[idxv.at[1]], sem)

    pltpu.sync_copy(idx_hbm, idx_all.at[pl.ds(0, _K)])
    sent = jnp.full((16,), _N, jnp.int32)
    idx_all[pl.ds(_K, 16)] = sent
    idx_all[pl.ds(_KPAD - 16, 16)] = sent

    pltpu.sync_copy(z_hbm, zbuf)

    nb = jnp.minimum(wid * _ZCH, _N - _ZCH)

    def _win(t, p):
        r = nb + t * 16 + lanes
        pos = jnp.zeros((16,), jnp.int32)
        for step in (4096, 2048, 1024, 512, 256, 128, 64, 32, 16, 8, 4, 2, 1):
            nxt = pos + step
            probe = jnp.minimum(nxt - 1, _KPAD - 1)
            v = plsc.load_gather(idx_all, [probe])
            pos = jnp.where(v < r, nxt, pos)
        g = plsc.load_gather(idx_all, [pos])
        notmem = g != r
        ones = jnp.where(notmem, 1, 0).astype(jnp.int32)
        inc = plsc.cumsum(ones)
        plsc.store_scatter(zlist1, [p + inc - 1], r, mask=notmem)
        return p + jnp.sum(ones)

    p = lax.fori_loop(0, _ZCH // 16, _win, jnp.int32(0))

    v0 = jnp.full((16,), 1, jnp.int32) * _sread(zlist1, 0)

    def _pad(t, _):
        w = zlist1[pl.ds(t * 16, 16)]
        gidx = t * 16 + lanes
        zlist1[pl.ds(t * 16, 16)] = jnp.where(gidx < p, w, v0)
        return 0

    lax.fori_loop(0, _ZCH // 16, _pad, 0)

    for c in range(_ZCH // _SUB):
        for u in range(_SUB // 16):
            zlist2[c, pl.ds(u * 16, 16)] = zlist1[pl.ds(c * _SUB + u * 16, 16)]

    for c in range(_ZCH // _SUB):
        @pl.when(p > c * _SUB)
        def _zs():
            pltpu.sync_copy(zbuf, out_hbm.at[zlist2.at[c]])

    cp0.wait()
    cp1.wait()


_mesh = plsc.VectorSubcoreMesh(
    core_axis_name="c", subcore_axis_name="s", num_cores=_NC, num_subcores=_NS
)
_unpool = pl.kernel(
    _unpool_body,
    out_type=jax.ShapeDtypeStruct((_N, _D), jnp.float32),
    mesh=_mesh,
    compiler_params=pltpu.CompilerParams(needs_layout_passes=False),
    scratch_types=[
        pltpu.VMEM((2, _SUB), jnp.int32),
        pltpu.VMEM((_CH, _D), jnp.float32),
        pltpu.VMEM((_KPAD,), jnp.int32),
        pltpu.VMEM((_SUB, _D), jnp.float32),
        pltpu.VMEM((_ZCH + 32, ), jnp.int32),
        pltpu.VMEM((_ZCH // _SUB, _SUB), jnp.int32),
        pltpu.SemaphoreType.DMA,
    ],
)

_CPROWS = 200


def _copy_body(a_ref, out_ref):
    out_ref[...] = a_ref[...]


_copy = pl.pallas_call(
    _copy_body,
    grid=(_N // _CPROWS,),
    in_specs=[pl.BlockSpec((_CPROWS, _N), lambda i: (i, 0))],
    out_specs=pl.BlockSpec((_CPROWS, _N), lambda i: (i, 0)),
    out_shape=jax.ShapeDtypeStruct((_N, _N), jnp.float32),
)


def kernel(A, X, idx):
    zrows = jnp.zeros((_SUB, _D), jnp.float32)
    new_X = _unpool(X, idx.astype(jnp.int32), zrows)
    return (_copy(A), new_X)


# speedup vs baseline: 1.0117x; 1.0117x over previous
import jax
import jax.numpy as jnp
from jax import lax
from jax.experimental import pallas as pl
from jax.experimental.pallas import tpu as pltpu
from jax.experimental.pallas import tpu_sc as plsc
from jax._src.pallas import mpmd as _mpmd

_N = 10000
_K = 5000
_D = 512

_NC = 2
_NS = 16
_NW = _NC * _NS
_CH = 160
_SUB = 80


def _scatter_body(zeros_hbm, x_hbm, idx_hbm, out_hbm, idxv, xv, sem):
    del zeros_hbm
    wid = lax.axis_index("s") * _NC + lax.axis_index("c")
    base = jnp.minimum(wid * _CH, _K - _CH)
    pltpu.sync_copy(idx_hbm.at[pl.ds(base, _SUB)], idxv.at[0])
    pltpu.sync_copy(idx_hbm.at[pl.ds(base + _SUB, _SUB)], idxv.at[1])
    pltpu.sync_copy(x_hbm.at[pl.ds(base, _CH)], xv)
    cp0 = pltpu.async_copy(xv.at[pl.ds(0, _SUB)], out_hbm.at[idxv.at[0]], sem)
    cp1 = pltpu.async_copy(xv.at[pl.ds(_SUB, _SUB)], out_hbm.at[idxv.at[1]], sem)
    cp0.wait()
    cp1.wait()


_mesh = plsc.VectorSubcoreMesh(
    core_axis_name="c", subcore_axis_name="s", num_cores=_NC, num_subcores=_NS
)
_scatter = _mpmd._mpmd_map(
    [(_mesh, _scatter_body)],
    jax.ShapeDtypeStruct((_N, _D), jnp.float32),
    input_output_aliases={0: 0},
    scratch_types=[
        pltpu.VMEM((2, _SUB), jnp.int32),
        pltpu.VMEM((_CH, _D), jnp.float32),
        pltpu.SemaphoreType.DMA,
    ],
)


_CPROWS = 400


def _copy_body(a_ref, out_ref):
    out_ref[...] = a_ref[...]


_copy = pl.pallas_call(
    _copy_body,
    grid=(_N // _CPROWS,),
    in_specs=[pl.BlockSpec((_CPROWS, _N), lambda i: (i, 0))],
    out_specs=pl.BlockSpec((_CPROWS, _N), lambda i: (i, 0)),
    out_shape=jax.ShapeDtypeStruct((_N, _N), jnp.float32),
    compiler_params=pltpu.CompilerParams(vmem_limit_bytes=100 * 1024 * 1024),
)


def kernel(A, X, idx):
    zeros = jnp.zeros((A.shape[0], X.shape[1]), dtype=X.dtype)
    new_X = _scatter(zeros, X, idx.astype(jnp.int32))
    return (_copy(A), new_X)
